# baseline (speedup 1.0000x reference)
import jax
import jax.numpy as jnp
from jax import lax
from jax.experimental import pallas as pl
from jax.experimental.pallas import tpu as pltpu
from jax.experimental.pallas import tpu_sc as plsc

_B = 16384
_D = 32
_NE = 1000000
_NC = 2
_NS = 16
_NW = _NC * _NS
_NCOL = _NE // 128
_NTAIL = _NE - _NCOL * 128
_NLINES = _NE * _D // 128
_BPW = _B // _NW
_NPIECE = 4
_PIECE = _BPW // _NPIECE
_NCHUNK = _PIECE // 16
_NRV = 1000
_RV_LINES = _NRV * _D // 128

_COL_BASE = _NCOL // _NW
_COL_REM = _NCOL % _NW


def _relayout(Et_hbm, tail_hbm, lines_hbm,
              blk0, blk1, out0, out1, tail_v,
              sem_i0, sem_i1, sem_o0, sem_o1):
    wid = lax.axis_index("s") * _NC + lax.axis_index("c")

    blks = (blk0, blk1)
    outs = (out0, out1)
    sem_is = (sem_i0, sem_i1)
    sem_os = (sem_o0, sem_o1)

    lanes = lax.iota(jnp.int32, 16)

    @pl.when(wid == _NW - 1)
    def _tail():
        pltpu.sync_copy(tail_hbm, tail_v)
        pltpu.sync_copy(tail_v, lines_hbm.at[pl.ds(_NCOL * _D, _NTAIL * _D // 128)])

    def col_of(k):
        return k * _NW + wid

    def fire_in(b, c):
        return pltpu.async_copy(
            Et_hbm.at[:, pl.ds(c * 128, 128)],
            blks[b].at[:, pl.ds(0, 128)], sem_is[b])

    def wait_in(b, c):
        pltpu.make_async_copy(
            Et_hbm.at[:, pl.ds(c * 128, 128)],
            blks[b].at[:, pl.ds(0, 128)], sem_is[b]).wait()

    def fire_out(b, c):
        return pltpu.async_copy(
            outs[b], lines_hbm.at[pl.ds(c * _D, _D)], sem_os[b])

    def wait_out(b, c):
        pltpu.make_async_copy(
            outs[b], lines_hbm.at[pl.ds(c * _D, _D)], sem_os[b]).wait()

    def transpose(b):
        blk = blks[b]
        out = outs[b]

        def lq_body(lq, carry):
            for half in range(8):
                d_vec = (half & 1) * 16 + lanes
                q = lq * 4 + (half >> 1)
                vals = plsc.load_gather(blk, [d_vec, jnp.full((16,), q, jnp.int32)])
                out[lq, pl.ds(half * 16, 16)] = vals
            return carry

        lax.fori_loop(0, _D, lq_body, 0)

    fire_in(0, col_of(0))
    fire_in(1, col_of(1))

    def pair_body(k2, carry):
        for b in range(2):
            k = k2 * 2 + b
            c = col_of(k)
            wait_in(b, c)

            @pl.when(k2 > 0)
            def _drain():
                wait_out(b, c)

            transpose(b)
            fire_out(b, c)

            @pl.when(k2 < _COL_BASE // 2 - 1)
            def _prefetch():
                fire_in(b, col_of(k + 2))
        return carry

    lax.fori_loop(0, _COL_BASE // 2, pair_body, 0)

    c_last0 = col_of(_COL_BASE - 2)
    c_last1 = col_of(_COL_BASE - 1)
    wait_out(0, c_last0)
    wait_out(1, c_last1)

    @pl.when(wid < _COL_REM)
    def _extra():
        c = _COL_BASE * _NW + wid
        pltpu.sync_copy(Et_hbm.at[:, pl.ds(c * 128, 128)],
                        blk0.at[:, pl.ds(0, 128)])
        transpose(0)
        pltpu.sync_copy(out0, lines_hbm.at[pl.ds(c * _D, _D)])


def _sc_score(lines_hbm, rv_hbm, u_hbm, r_hbm, v_hbm, out_hbm,
              u_idx_v, r_idx_v, v_idx_v, u_line_v, v_line_v,
              rv_l, u_l0, u_l1, v_l0, v_l1, out_v,
              sem_rv, sem_u0, sem_u1, sem_v0, sem_v1):
    wid = lax.axis_index("s") * _NC + lax.axis_index("c")
    base = wid * _BPW

    crv = pltpu.async_copy(rv_hbm, rv_l, sem_rv)

    pltpu.sync_copy(u_hbm.at[pl.ds(base, _BPW)], u_idx_v)
    pltpu.sync_copy(v_hbm.at[pl.ds(base, _BPW)], v_idx_v)
    pltpu.sync_copy(r_hbm.at[pl.ds(base, _BPW)], r_idx_v)

    for s in range(_BPW // 16):
        sl = pl.ds(s * 16, 16)
        u_line_v[sl] = lax.shift_right_logical(u_idx_v[sl], 2)
        v_line_v[sl] = lax.shift_right_logical(v_idx_v[sl], 2)

    u_bufs = (u_l0, u_l1)
    v_bufs = (v_l0, v_l1)
    u_sems = (sem_u0, sem_u1)
    v_sems = (sem_v0, sem_v1)

    def fire(p):
        sl = pl.ds(p * _PIECE, _PIECE)
        cu = pltpu.async_copy(lines_hbm.at[u_line_v.at[sl]], u_bufs[p % 2],
                              u_sems[p % 2])
        cv = pltpu.async_copy(lines_hbm.at[v_line_v.at[sl]], v_bufs[p % 2],
                              v_sems[p % 2])
        return cu, cv

    lanes = lax.iota(jnp.int32, 16)
    three = jnp.full((16,), 3, jnp.int32)

    pend = fire(0)
    crv.wait()

    for p in range(_NPIECE):
        pend[0].wait()
        pend[1].wait()
        if p + 1 < _NPIECE:
            pend = fire(p + 1)
        u_buf = u_bufs[p % 2]
        v_buf = v_bufs[p % 2]

        def chunk_body(c, carry, p=p, u_buf=u_buf, v_buf=v_buf):
            b0 = p * _PIECE + c * 16
            rowloc = c * 16 + lanes
            uidx = u_idx_v[pl.ds(b0, 16)]
            vidx = v_idx_v[pl.ds(b0, 16)]
            ridx = r_idx_v[pl.ds(b0, 16)]
            usub = lax.shift_left(jnp.bitwise_and(uidx, three), 5)
            vsub = lax.shift_left(jnp.bitwise_and(vidx, three), 5)
            rline = lax.shift_right_logical(ridx, 2)
            rsub = lax.shift_left(jnp.bitwise_and(ridx, three), 5)
            acc = jnp.zeros((16,), jnp.float32)
            for d in range(_D):
                ud = plsc.load_gather(u_buf, [rowloc, usub + d])
                vd = plsc.load_gather(v_buf, [rowloc, vsub + d])
                rd = plsc.load_gather(rv_l, [rline, rsub + d])
                t = ud - (vd + rd)
                acc = acc + t * t
            out_v[pl.ds(b0, 16)] = -acc
            return carry

        lax.fori_loop(0, _NCHUNK, chunk_body, 0)

    pltpu.sync_copy(out_v, out_hbm.at[pl.ds(base, _BPW)])


@jax.jit
def kernel(E, rv, u_idx, r_idx, v_idx):
    Et = E.T
    tail = E[_NCOL * 128:, :].reshape(_NTAIL * _D // 128, 128)
    rv_lines = rv.reshape(-1, 128)
    mesh = plsc.VectorSubcoreMesh(core_axis_name="c", subcore_axis_name="s")

    relayout = pl.kernel(
        _relayout,
        out_type=jax.ShapeDtypeStruct((_NLINES, 128), jnp.float32),
        mesh=mesh,
        compiler_params=pltpu.CompilerParams(needs_layout_passes=False),
        scratch_types=[
            pltpu.VMEM((_D, 129), jnp.float32),
            pltpu.VMEM((_D, 129), jnp.float32),
            pltpu.VMEM((_D, 128), jnp.float32),
            pltpu.VMEM((_D, 128), jnp.float32),
            pltpu.VMEM((_NTAIL * _D // 128, 128), jnp.float32),
            pltpu.SemaphoreType.DMA,
            pltpu.SemaphoreType.DMA,
            pltpu.SemaphoreType.DMA,
            pltpu.SemaphoreType.DMA,
        ],
    )
    lines = relayout(Et, tail)

    score = pl.kernel(
        _sc_score,
        out_type=jax.ShapeDtypeStruct((_B,), jnp.float32),
        mesh=mesh,
        compiler_params=pltpu.CompilerParams(needs_layout_passes=False),
        scratch_types=[
            pltpu.VMEM((_BPW,), jnp.int32),
            pltpu.VMEM((_BPW,), jnp.int32),
            pltpu.VMEM((_BPW,), jnp.int32),
            pltpu.VMEM((_BPW,), jnp.int32),
            pltpu.VMEM((_BPW,), jnp.int32),
            pltpu.VMEM((_RV_LINES, 128), jnp.float32),
            pltpu.VMEM((_PIECE, 128), jnp.float32),
            pltpu.VMEM((_PIECE, 128), jnp.float32),
            pltpu.VMEM((_PIECE, 128), jnp.float32),
            pltpu.VMEM((_PIECE, 128), jnp.float32),
            pltpu.VMEM((_BPW,), jnp.float32),
            pltpu.SemaphoreType.DMA,
            pltpu.SemaphoreType.DMA,
            pltpu.SemaphoreType.DMA,
            pltpu.SemaphoreType.DMA,
            pltpu.SemaphoreType.DMA,
        ],
    )
    return score(lines, rv_lines, u_idx, r_idx, v_idx)


# speedup vs baseline: 5.4977x; 5.4977x over previous
import jax
import jax.numpy as jnp
from jax import lax
from jax.experimental import pallas as pl
from jax.experimental.pallas import tpu as pltpu
from jax.experimental.pallas import tpu_sc as plsc

_B = 16384
_D = 32
_NE = 1000000
_NC = 2
_NS = 16
_DPC = _D // _NC
_IPT = _B // _NS
_NRV = 1000
_RV_LINES = _NRV * _D // 128
_NW = _NC * _NS
_BPW = _B // _NW


def _partial(Et_hbm, rv_hbm, u_hbm, r_hbm, v_hbm, part_hbm,
             row_sh, u_idx_v, r_idx_v, v_idx_v, u_val, v_val, acc, rv_l,
             sem_rv, sem_u, sem_v):
    cid = lax.axis_index("c")
    sid = lax.axis_index("s")
    base = sid * _IPT

    crv = pltpu.async_copy(rv_hbm, rv_l, sem_rv)
    pltpu.sync_copy(u_hbm.at[pl.ds(base, _IPT)], u_idx_v)
    pltpu.sync_copy(v_hbm.at[pl.ds(base, _IPT)], v_idx_v)
    pltpu.sync_copy(r_hbm.at[pl.ds(base, _IPT)], r_idx_v)

    for s in range(_IPT // 16):
        acc[pl.ds(s * 16, 16)] = jnp.zeros((16,), jnp.float32)
    crv.wait()

    three = jnp.full((16,), 3, jnp.int32)

    def dim_body(k, carry):
        d = cid * _DPC + k

        @pl.when(sid == 0)
        def _stage():
            pltpu.sync_copy(Et_hbm.at[d], row_sh)

        plsc.subcore_barrier()

        cu = pltpu.async_copy(row_sh.at[u_idx_v], u_val, sem_u)
        cv = pltpu.async_copy(row_sh.at[v_idx_v], v_val, sem_v)
        cu.wait()
        cv.wait()

        def slice_body(s, carry2):
            sl = pl.ds(s * 16, 16)
            ridx = r_idx_v[sl]
            rline = lax.shift_right_logical(ridx, 2)
            rcol = lax.shift_left(jnp.bitwise_and(ridx, three), 5) + d
            rd = plsc.load_gather(rv_l, [rline, rcol])
            t = u_val[sl] - (v_val[sl] + rd)
            acc[sl] = acc[sl] + t * t
            return carry2

        lax.fori_loop(0, _IPT // 16, slice_body, 0)
        plsc.subcore_barrier()
        return carry

    lax.fori_loop(0, _DPC, dim_body, 0)

    pltpu.sync_copy(acc, part_hbm.at[cid, pl.ds(base, _IPT)])


def _combine(part_hbm, out_hbm, p0_v, p1_v, out_v):
    wid = lax.axis_index("s") * _NC + lax.axis_index("c")
    base = wid * _BPW
    pltpu.sync_copy(part_hbm.at[0, pl.ds(base, _BPW)], p0_v)
    pltpu.sync_copy(part_hbm.at[1, pl.ds(base, _BPW)], p1_v)
    for s in range(_BPW // 16):
        sl = pl.ds(s * 16, 16)
        out_v[sl] = -(p0_v[sl] + p1_v[sl])
    pltpu.sync_copy(out_v, out_hbm.at[pl.ds(base, _BPW)])


@jax.jit
def kernel(E, rv, u_idx, r_idx, v_idx):
    Et = E.T
    rv_lines = rv.reshape(-1, 128)
    mesh = plsc.VectorSubcoreMesh(core_axis_name="c", subcore_axis_name="s")

    partial = pl.kernel(
        _partial,
        out_type=jax.ShapeDtypeStruct((_NC, _B), jnp.float32),
        mesh=mesh,
        compiler_params=pltpu.CompilerParams(needs_layout_passes=False),
        scratch_types=[
            pltpu.VMEM_SHARED((_NE,), jnp.float32),
            pltpu.VMEM((_IPT,), jnp.int32),
            pltpu.VMEM((_IPT,), jnp.int32),
            pltpu.VMEM((_IPT,), jnp.int32),
            pltpu.VMEM((_IPT,), jnp.float32),
            pltpu.VMEM((_IPT,), jnp.float32),
            pltpu.VMEM((_IPT,), jnp.float32),
            pltpu.VMEM((_RV_LINES, 128), jnp.float32),
            pltpu.SemaphoreType.DMA,
            pltpu.SemaphoreType.DMA,
            pltpu.SemaphoreType.DMA,
        ],
    )
    part = partial(Et, rv_lines, u_idx, r_idx, v_idx)

    combine = pl.kernel(
        _combine,
        out_type=jax.ShapeDtypeStruct((_B,), jnp.float32),
        mesh=mesh,
        compiler_params=pltpu.CompilerParams(needs_layout_passes=False),
        scratch_types=[
            pltpu.VMEM((_BPW,), jnp.float32),
            pltpu.VMEM((_BPW,), jnp.float32),
            pltpu.VMEM((_BPW,), jnp.float32),
        ],
    )
    return combine(part)
